# Initial kernel scaffold; baseline (speedup 1.0000x reference)
import jax
import jax.numpy as jnp
from jax.experimental import pallas as pl


def kernel(xyz, x, Wqk, bqk, Wv, bv, Wd1, bd1, Wd2, bd2):
    raise NotImplementedError("write your pallas kernel here")


# speedup vs baseline: 12.6514x; 12.6514x over previous
import functools

import jax
import jax.numpy as jnp
from jax.experimental import pallas as pl
from jax.experimental.pallas import tpu as pltpu
from jax.experimental.pallas import tpu_sc as plsc

KNN = 16
PADC = 16
TN_A = 256
TN_C = 128
TN_D = 256


def _topk_proj_body(n_points, xyzp_tile, xyzp_full, x_tile, wqk, bqk, wv, bv,
                    wd1, table_out, idx_out):
    b = pl.program_id(0)
    tn = xyzp_tile.shape[1]
    xt = xyzp_tile[0]
    xf = xyzp_full[0]
    rn = jnp.sum(xt * xt, axis=1, keepdims=True)
    fn = jnp.sum(xf * xf, axis=1, keepdims=True)
    xt_aug = jnp.concatenate([-2.0 * xt, jnp.ones((tn, 1), jnp.float32)],
                             axis=1)
    xf_aug = jnp.concatenate([xf, fn], axis=1)
    d = jax.lax.dot_general(xt_aug, xf_aug, (((1,), (1,)), ((), ())),
                            preferred_element_type=jnp.float32)
    d = d + rn

    col = jax.lax.broadcasted_iota(jnp.int32, (tn, n_points), 1)
    sels = []
    for _ in range(KNN):
        m = jnp.min(d, axis=1, keepdims=True)
        cand = jnp.where(d == m, col, n_points)
        sel = jnp.min(cand, axis=1, keepdims=True)
        sels.append(sel)
        d = jnp.where(col == sel, jnp.inf, d)
    idx_tile = jnp.concatenate(sels, axis=1)
    idx_out[...] = idx_tile + b * n_points

    xx = x_tile[0]
    q = jnp.dot(xx, wqk[...], preferred_element_type=jnp.float32) + bqk[...]
    v = jnp.dot(xx, wv[...], preferred_element_type=jnp.float32) + bv[...]
    p = jnp.dot(xt, wd1[...], preferred_element_type=jnp.float32)
    table_out[...] = jnp.concatenate([q * q, v, p], axis=1)


def _attn_body(tn_c, g_ref, tblq_ref, bd1_ref, wd2_ref, bd2_ref, t_ref, s_ref):
    i = pl.program_id(1)
    g = g_ref[...]
    c = g.shape[1] // 3
    xq2 = g[:, :c]
    xv = g[:, c:2 * c]
    pg = g[:, 2 * c:]
    pq = tblq_ref[...][:, 2 * c:]
    pqb = jnp.broadcast_to(pq[:, None, :], (tn_c, KNN, c)).reshape(
        tn_c * KNN, c)
    h = pqb - pg + bd1_ref[...]
    pos = jnp.dot(jnp.maximum(h, 0.0), wd2_ref[...],
                  preferred_element_type=jnp.float32) + bd2_ref[...]
    energy = pos + xq2
    mx = jnp.max(energy, axis=1, keepdims=True)
    e = jnp.exp(energy - mx)
    attn = e / jnp.sum(e, axis=1, keepdims=True)
    t_ref[...] = attn * xv
    s_part = jnp.sum(attn.reshape(tn_c, KNN, c), axis=0)

    @pl.when(i == 0)
    def _init():
        s_ref[0] = s_part

    @pl.when(i > 0)
    def _acc():
        s_ref[0] = s_ref[0] + s_part


def _norm_body(tn_d, t_ref, s_ref, x_ref, out_ref):
    c = x_ref.shape[2]
    sinv = 1.0 / (1e-9 + s_ref[0])
    t = t_ref[...].reshape(tn_d, KNN, c)
    out_ref[0] = x_ref[0] + jnp.sum(t * sinv[None, :, :], axis=1)


def _sc_gather(table, idx_flat):
    n_idx = idx_flat.shape[1]
    dbig = table.shape[1]
    win = 128
    mesh = plsc.VectorSubcoreMesh(core_axis_name="c", subcore_axis_name="s")

    @functools.partial(
        pl.kernel,
        out_type=jax.ShapeDtypeStruct((n_idx, dbig), jnp.float32),
        mesh=mesh,
    )
    def sc_kernel(tab_hbm, i_hbm, g_hbm):
        def body(i_vmem, g_vmem):
            pltpu.sync_copy(tab_hbm.at[i_vmem.at[0]], g_vmem)

        pltpu.emit_pipeline(
            body,
            grid=(n_idx // win,),
            in_specs=[pl.BlockSpec((1, win), lambda i: (0, i))],
            out_specs=[pl.BlockSpec((win, dbig), lambda i: (i, 0))],
            core_axis_name=("c", "s"),
            dimension_semantics=(pltpu.PARALLEL,),
        )(i_hbm, g_hbm)

    return sc_kernel(table, idx_flat)


@jax.jit
def kernel(xyz, x, Wqk, bqk, Wv, bv, Wd1, bd1, Wd2, bd2):
    B, N, C = x.shape
    f32 = jnp.float32

    xyzp = jnp.pad(xyz, ((0, 0), (0, 0), (0, PADC - xyz.shape[2])))
    wd1p = jnp.pad(Wd1, ((0, PADC - Wd1.shape[0]), (0, 0)))
    bqk2 = bqk.reshape(1, C)
    bv2 = bv.reshape(1, C)
    bd12 = bd1.reshape(1, C)
    bd22 = bd2.reshape(1, C)

    n_tiles_a = N // TN_A
    table, idx = pl.pallas_call(
        functools.partial(_topk_proj_body, N),
        grid=(B, n_tiles_a),
        in_specs=[
            pl.BlockSpec((1, TN_A, PADC), lambda b, i: (b, i, 0)),
            pl.BlockSpec((1, N, PADC), lambda b, i: (b, 0, 0)),
            pl.BlockSpec((1, TN_A, C), lambda b, i: (b, i, 0)),
            pl.BlockSpec((C, C), lambda b, i: (0, 0)),
            pl.BlockSpec((1, C), lambda b, i: (0, 0)),
            pl.BlockSpec((C, C), lambda b, i: (0, 0)),
            pl.BlockSpec((1, C), lambda b, i: (0, 0)),
            pl.BlockSpec((PADC, C), lambda b, i: (0, 0)),
        ],
        out_specs=[
            pl.BlockSpec((TN_A, 3 * C), lambda b, i: (b * (N // TN_A) + i, 0)),
            pl.BlockSpec((TN_A, KNN), lambda b, i: (b * (N // TN_A) + i, 0)),
        ],
        out_shape=[
            jax.ShapeDtypeStruct((B * N, 3 * C), f32),
            jax.ShapeDtypeStruct((B * N, KNN), jnp.int32),
        ],
    )(xyzp, xyzp, x, Wqk, bqk2, Wv, bv2, wd1p)

    idx_flat = idx.reshape(1, B * N * KNN)
    g = _sc_gather(table, idx_flat)

    n_tiles_c = N // TN_C
    t_arr, s_arr = pl.pallas_call(
        functools.partial(_attn_body, TN_C),
        grid=(B, n_tiles_c),
        in_specs=[
            pl.BlockSpec((TN_C * KNN, 3 * C),
                         lambda b, i: (b * (N // TN_C) + i, 0)),
            pl.BlockSpec((TN_C, 3 * C),
                         lambda b, i: (b * (N // TN_C) + i, 0)),
            pl.BlockSpec((1, C), lambda b, i: (0, 0)),
            pl.BlockSpec((C, C), lambda b, i: (0, 0)),
            pl.BlockSpec((1, C), lambda b, i: (0, 0)),
        ],
        out_specs=[
            pl.BlockSpec((TN_C * KNN, C), lambda b, i: (b * (N // TN_C) + i, 0)),
            pl.BlockSpec((1, KNN, C), lambda b, i: (b, 0, 0)),
        ],
        out_shape=[
            jax.ShapeDtypeStruct((B * N * KNN, C), f32),
            jax.ShapeDtypeStruct((B, KNN, C), f32),
        ],
    )(g, table, bd12, Wd2, bd22)

    n_tiles_d = N // TN_D
    out = pl.pallas_call(
        functools.partial(_norm_body, TN_D),
        grid=(B, n_tiles_d),
        in_specs=[
            pl.BlockSpec((TN_D * KNN, C),
                         lambda b, i: (b * (N // TN_D) + i, 0)),
            pl.BlockSpec((1, KNN, C), lambda b, i: (b, 0, 0)),
            pl.BlockSpec((1, TN_D, C), lambda b, i: (b, i, 0)),
        ],
        out_specs=pl.BlockSpec((1, TN_D, C), lambda b, i: (b, i, 0)),
        out_shape=jax.ShapeDtypeStruct((B, N, C), f32),
    )(t_arr, s_arr, x)

    return out


# speedup vs baseline: 17.2906x; 1.3667x over previous
import functools

import jax
import jax.numpy as jnp
from jax.experimental import pallas as pl
from jax.experimental.pallas import tpu as pltpu
from jax.experimental.pallas import tpu_sc as plsc

KNN = 16
PADC = 16
TN_A = 256
TN_C = 128
TN_D = 256


def _topk_proj_body(n_points, xyzp_tile, xyzp_full, x_tile,
                    wqk, bqk, wv, bv, wd1, table_out, idx_out):
    tn = xyzp_tile.shape[0]
    xt = xyzp_tile[...]
    xf = xyzp_full[...]
    rn = jnp.sum(xt * xt, axis=1, keepdims=True)
    fn = jnp.sum(xf * xf, axis=1, keepdims=True)
    xt_aug = jnp.concatenate([-2.0 * xt, jnp.ones((tn, 1), jnp.float32)],
                             axis=1)
    xf_aug = jnp.concatenate([xf, fn], axis=1)
    d = jax.lax.dot_general(xt_aug, xf_aug, (((1,), (1,)), ((), ())),
                            preferred_element_type=jnp.float32)
    d = d + rn

    col_f = jax.lax.broadcasted_iota(jnp.int32, (tn, n_points), 1).astype(
        jnp.float32)
    big = jnp.float32(n_points)
    sels = []
    for j in range(KNN):
        m = jnp.min(d, axis=1, keepdims=True)
        cand = jnp.where(d == m, col_f, big)
        sel = jnp.min(cand, axis=1, keepdims=True)
        sels.append(sel)
        if j + 1 < KNN:
            d = jnp.where(col_f == sel, jnp.inf, d)
    idx_out[...] = jnp.concatenate(sels, axis=1).astype(jnp.int32)

    xx = x_tile[...]
    q = jnp.dot(xx, wqk[...], preferred_element_type=jnp.float32) + bqk[...]
    v = jnp.dot(xx, wv[...], preferred_element_type=jnp.float32) + bv[...]
    p = jnp.dot(xt, wd1[...], preferred_element_type=jnp.float32)
    table_out[...] = jnp.concatenate([q * q, v, p], axis=1)


def _attn_body(tn_c, g_ref, tblq_ref, bd1_ref, wd2_ref, bd2_ref, t_ref, s_ref):
    i = pl.program_id(0)
    g = g_ref[...]
    c = g.shape[1] // 3
    xq2 = g[:, :c]
    xv = g[:, c:2 * c]
    pg = g[:, 2 * c:]
    pq = tblq_ref[...][:, 2 * c:]
    pqb = jnp.broadcast_to(pq[:, None, :], (tn_c, KNN, c)).reshape(
        tn_c * KNN, c)
    h = pqb - pg + bd1_ref[...]
    pos = jnp.dot(jnp.maximum(h, 0.0), wd2_ref[...],
                  preferred_element_type=jnp.float32) + bd2_ref[...]
    energy = pos + xq2
    mx = jnp.max(energy, axis=1, keepdims=True)
    e = jnp.exp(energy - mx)
    attn = e / jnp.sum(e, axis=1, keepdims=True)
    t_ref[...] = attn * xv
    s_part = jnp.sum(attn.reshape(tn_c, KNN, c), axis=0)

    @pl.when(i == 0)
    def _init():
        s_ref[...] = s_part

    @pl.when(i > 0)
    def _acc():
        s_ref[...] = s_ref[...] + s_part


def _norm_body(tn_d, t_ref, s_ref, x_ref, out_ref):
    c = x_ref.shape[1]
    sinv = 1.0 / (1e-9 + s_ref[...])
    t = t_ref[...].reshape(tn_d, KNN, c)
    out_ref[...] = x_ref[...] + jnp.sum(t * sinv[None, :, :], axis=1)


def _sc_gather(table, idx_flat):
    n_idx = idx_flat.shape[1]
    dbig = table.shape[1]
    win = 128
    mesh = plsc.VectorSubcoreMesh(core_axis_name="c", subcore_axis_name="s")

    @functools.partial(
        pl.kernel,
        out_type=jax.ShapeDtypeStruct((n_idx, dbig), jnp.float32),
        mesh=mesh,
    )
    def sc_kernel(tab_hbm, i_hbm, g_hbm):
        def body(i_vmem, g_vmem):
            pltpu.sync_copy(tab_hbm.at[i_vmem.at[0]], g_vmem)

        pltpu.emit_pipeline(
            body,
            grid=(n_idx // win,),
            in_specs=[pl.BlockSpec((1, win), lambda i: (0, i))],
            out_specs=[pl.BlockSpec((win, dbig), lambda i: (i, 0))],
            core_axis_name=("c", "s"),
            dimension_semantics=(pltpu.PARALLEL,),
        )(i_hbm, g_hbm)

    return sc_kernel(table, idx_flat)


@jax.jit
def kernel(xyz, x, Wqk, bqk, Wv, bv, Wd1, bd1, Wd2, bd2):
    B, N, C = x.shape
    f32 = jnp.float32

    xyzp = jnp.pad(xyz, ((0, 0), (0, 0), (0, PADC - xyz.shape[2])))
    wd1p = jnp.pad(Wd1, ((0, PADC - Wd1.shape[0]), (0, 0)))
    bqk2 = bqk.reshape(1, C)
    bv2 = bv.reshape(1, C)
    bd12 = bd1.reshape(1, C)
    bd22 = bd2.reshape(1, C)

    topk_call = pl.pallas_call(
        functools.partial(_topk_proj_body, N),
        grid=(N // TN_A,),
        in_specs=[
            pl.BlockSpec((TN_A, PADC), lambda i: (i, 0)),
            pl.BlockSpec((N, PADC), lambda i: (0, 0)),
            pl.BlockSpec((TN_A, C), lambda i: (i, 0)),
            pl.BlockSpec((C, C), lambda i: (0, 0)),
            pl.BlockSpec((1, C), lambda i: (0, 0)),
            pl.BlockSpec((C, C), lambda i: (0, 0)),
            pl.BlockSpec((1, C), lambda i: (0, 0)),
            pl.BlockSpec((PADC, C), lambda i: (0, 0)),
        ],
        out_specs=[
            pl.BlockSpec((TN_A, 3 * C), lambda i: (i, 0)),
            pl.BlockSpec((TN_A, KNN), lambda i: (i, 0)),
        ],
        out_shape=[
            jax.ShapeDtypeStruct((N, 3 * C), f32),
            jax.ShapeDtypeStruct((N, KNN), jnp.int32),
        ],
    )

    attn_call = pl.pallas_call(
        functools.partial(_attn_body, TN_C),
        grid=(N // TN_C,),
        in_specs=[
            pl.BlockSpec((TN_C * KNN, 3 * C), lambda i: (i, 0)),
            pl.BlockSpec((TN_C, 3 * C), lambda i: (i, 0)),
            pl.BlockSpec((1, C), lambda i: (0, 0)),
            pl.BlockSpec((C, C), lambda i: (0, 0)),
            pl.BlockSpec((1, C), lambda i: (0, 0)),
        ],
        out_specs=[
            pl.BlockSpec((TN_C * KNN, C), lambda i: (i, 0)),
            pl.BlockSpec((KNN, C), lambda i: (0, 0)),
        ],
        out_shape=[
            jax.ShapeDtypeStruct((N * KNN, C), f32),
            jax.ShapeDtypeStruct((KNN, C), f32),
        ],
    )

    norm_call = pl.pallas_call(
        functools.partial(_norm_body, TN_D),
        grid=(N // TN_D,),
        in_specs=[
            pl.BlockSpec((TN_D * KNN, C), lambda i: (i, 0)),
            pl.BlockSpec((KNN, C), lambda i: (0, 0)),
            pl.BlockSpec((TN_D, C), lambda i: (i, 0)),
        ],
        out_specs=pl.BlockSpec((TN_D, C), lambda i: (i, 0)),
        out_shape=jax.ShapeDtypeStruct((N, C), f32),
    )

    outs = []
    for b in range(B):
        table, idx = topk_call(xyzp[b], xyzp[b], x[b], Wqk, bqk2, Wv, bv2,
                               wd1p)
        g = _sc_gather(table, idx.reshape(1, N * KNN))
        t_arr, s_arr = attn_call(g, table, bd12, Wd2, bd22)
        outs.append(norm_call(t_arr, s_arr, x[b]))
    return jnp.stack(outs)


# speedup vs baseline: 17.2946x; 1.0002x over previous
import functools

import jax
import jax.numpy as jnp
from jax.experimental import pallas as pl
from jax.experimental.pallas import tpu as pltpu
from jax.experimental.pallas import tpu_sc as plsc

KNN = 16
PADC = 16
TN_A = 256
TN_C = 128
TN_D = 256


def _topk_proj_body(n_points, xyzp_tile, xyzp_full, x_tile,
                    wqk, bqk, wv, bv, wd1, table_out, idx_out):
    tn = xyzp_tile.shape[0]
    xt = xyzp_tile[...]
    xf = xyzp_full[...]
    rn = jnp.sum(xt * xt, axis=1, keepdims=True)
    fn = jnp.sum(xf * xf, axis=1, keepdims=True)
    xt_aug = jnp.concatenate([-2.0 * xt, jnp.ones((tn, 1), jnp.float32)],
                             axis=1)
    xf_aug = jnp.concatenate([xf, fn], axis=1)
    d = jax.lax.dot_general(xt_aug, xf_aug, (((1,), (1,)), ((), ())),
                            preferred_element_type=jnp.float32)
    d = d + rn

    col_f = jax.lax.broadcasted_iota(jnp.int32, (tn, n_points), 1).astype(
        jnp.float32)
    big = jnp.float32(n_points)
    sels = []
    for j in range(KNN):
        m = jnp.min(d, axis=1, keepdims=True)
        cand = jnp.where(d == m, col_f, big)
        sel = jnp.min(cand, axis=1, keepdims=True)
        sels.append(sel)
        if j + 1 < KNN:
            d = jnp.where(col_f == sel, jnp.inf, d)
    idx_out[...] = jnp.concatenate(sels, axis=1).astype(jnp.int32)

    xx = x_tile[...]
    q = jnp.dot(xx, wqk[...], preferred_element_type=jnp.float32) + bqk[...]
    v = jnp.dot(xx, wv[...], preferred_element_type=jnp.float32) + bv[...]
    p = jnp.dot(xt, wd1[...], preferred_element_type=jnp.float32)
    table_out[...] = jnp.concatenate([q * q, v, p], axis=1)


def _attn_body(tn_c, g_ref, tblq_ref, bd1_ref, wd2_ref, bd2_ref, t_ref, s_ref):
    i = pl.program_id(0)
    g = g_ref[...]
    c = g.shape[1] // 3
    xq2 = g[:, :c]
    xv = g[:, c:2 * c]
    pg = g[:, 2 * c:]
    pq = tblq_ref[...][:, 2 * c:]
    pqb = jnp.broadcast_to(pq[:, None, :], (tn_c, KNN, c)).reshape(
        tn_c * KNN, c)
    h = pqb - pg + bd1_ref[...]
    pos = jnp.dot(jnp.maximum(h, 0.0), wd2_ref[...],
                  preferred_element_type=jnp.float32) + bd2_ref[...]
    energy = pos + xq2
    mx = jnp.max(energy, axis=1, keepdims=True)
    e = jnp.exp(energy - mx)
    attn = e / jnp.sum(e, axis=1, keepdims=True)
    t_ref[...] = attn * xv
    s_part = jnp.sum(attn.reshape(tn_c, KNN, c), axis=0)

    @pl.when(i == 0)
    def _init():
        s_ref[...] = s_part

    @pl.when(i > 0)
    def _acc():
        s_ref[...] = s_ref[...] + s_part


def _norm_body(tn_d, t_ref, s_ref, x_ref, out_ref):
    c = x_ref.shape[1]
    sinv = 1.0 / (1e-9 + s_ref[...])
    t = t_ref[...].reshape(tn_d, KNN, c)
    out_ref[...] = x_ref[...] + jnp.sum(t * sinv[None, :, :], axis=1)


def _sc_gather(table, idx_flat):
    n_idx = idx_flat.shape[1]
    dbig = table.shape[1]
    win = 128
    mesh = plsc.VectorSubcoreMesh(core_axis_name="c", subcore_axis_name="s")

    @functools.partial(
        pl.kernel,
        out_type=jax.ShapeDtypeStruct((n_idx, dbig), jnp.float32),
        mesh=mesh,
    )
    def sc_kernel(tab_hbm, i_hbm, g_hbm):
        def body(i_vmem, g_vmem):
            pltpu.sync_copy(tab_hbm.at[i_vmem.at[0]], g_vmem)

        pltpu.emit_pipeline(
            body,
            grid=(n_idx // win,),
            in_specs=[pl.BlockSpec((1, win), lambda i: (0, i))],
            out_specs=[pl.BlockSpec((win, dbig), lambda i: (i, 0))],
            core_axis_name=("c", "s"),
            dimension_semantics=(pltpu.PARALLEL,),
        )(i_hbm, g_hbm)

    return sc_kernel(table, idx_flat)


@jax.jit
def kernel(xyz, x, Wqk, bqk, Wv, bv, Wd1, bd1, Wd2, bd2):
    B, N, C = x.shape
    f32 = jnp.float32

    xyzp = jnp.pad(xyz, ((0, 0), (0, 0), (0, PADC - xyz.shape[2])))
    wd1p = jnp.pad(Wd1, ((0, PADC - Wd1.shape[0]), (0, 0)))
    bqk2 = bqk.reshape(1, C)
    bv2 = bv.reshape(1, C)
    bd12 = bd1.reshape(1, C)
    bd22 = bd2.reshape(1, C)

    topk_call = pl.pallas_call(
        functools.partial(_topk_proj_body, N),
        grid=(N // TN_A,),
        in_specs=[
            pl.BlockSpec((TN_A, PADC), lambda i: (i, 0)),
            pl.BlockSpec((N, PADC), lambda i: (0, 0)),
            pl.BlockSpec((TN_A, C), lambda i: (i, 0)),
            pl.BlockSpec((C, C), lambda i: (0, 0)),
            pl.BlockSpec((1, C), lambda i: (0, 0)),
            pl.BlockSpec((C, C), lambda i: (0, 0)),
            pl.BlockSpec((1, C), lambda i: (0, 0)),
            pl.BlockSpec((PADC, C), lambda i: (0, 0)),
        ],
        out_specs=[
            pl.BlockSpec((TN_A, 3 * C), lambda i: (i, 0)),
            pl.BlockSpec((TN_A, KNN), lambda i: (i, 0)),
        ],
        out_shape=[
            jax.ShapeDtypeStruct((N, 3 * C), f32),
            jax.ShapeDtypeStruct((N, KNN), jnp.int32),
        ],
    )

    attn_call = pl.pallas_call(
        functools.partial(_attn_body, TN_C),
        grid=(N // TN_C,),
        in_specs=[
            pl.BlockSpec((TN_C * KNN, 3 * C), lambda i: (i, 0)),
            pl.BlockSpec((TN_C, 3 * C), lambda i: (i, 0)),
            pl.BlockSpec((1, C), lambda i: (0, 0)),
            pl.BlockSpec((C, C), lambda i: (0, 0)),
            pl.BlockSpec((1, C), lambda i: (0, 0)),
        ],
        out_specs=[
            pl.BlockSpec((TN_C * KNN, C), lambda i: (i, 0)),
            pl.BlockSpec((KNN, C), lambda i: (0, 0)),
        ],
        out_shape=[
            jax.ShapeDtypeStruct((N * KNN, C), f32),
            jax.ShapeDtypeStruct((KNN, C), f32),
        ],
    )

    norm_call = pl.pallas_call(
        functools.partial(_norm_body, TN_D),
        grid=(N // TN_D,),
        in_specs=[
            pl.BlockSpec((TN_D * KNN, C), lambda i: (i, 0)),
            pl.BlockSpec((KNN, C), lambda i: (0, 0)),
            pl.BlockSpec((TN_D, C), lambda i: (i, 0)),
        ],
        out_specs=pl.BlockSpec((TN_D, C), lambda i: (i, 0)),
        out_shape=jax.ShapeDtypeStruct((N, C), f32),
    )

    tables, idxs = [], []
    for b in range(B):
        table, idx = topk_call(xyzp[b], xyzp[b], x[b], Wqk, bqk2, Wv, bv2,
                               wd1p)
        tables.append(table)
        idxs.append(idx)
    gs = [_sc_gather(tables[b], idxs[b].reshape(1, N * KNN))
          for b in range(B)]
    outs = []
    for b in range(B):
        t_arr, s_arr = attn_call(gs[b], tables[b], bd12, Wd2, bd22)
        outs.append(norm_call(t_arr, s_arr, x[b]))
    return jnp.stack(outs)
